# baseline (speedup 1.0000x reference)
import functools

import jax
import jax.numpy as jnp
from jax import lax
from jax.experimental import pallas as pl
from jax.experimental.pallas import tpu as pltpu
from jax.experimental.pallas import tpu_sc as plsc

N = 10000
E = 320000
D = 128
H = 8
C = 16
DW = D + 16

NP = 10240
CHUNK = 64
NBUF = 3
NSLOT = 6
EPT0 = 10368
EPT1 = 10368
EPAD = 16 * (EPT0 + EPT1)
RPT = NP // 16

_f32 = jnp.float32


def _emb_body(x_ref, wt_ref, al_ref, ar_ref, ext_ref, r_ref):
    emb = jnp.dot(x_ref[...], wt_ref[...], preferred_element_type=_f32)
    ext_ref[:, :D] = emb
    ext_ref[:, D:] = jnp.dot(emb, al_ref[...], preferred_element_type=_f32)
    r_ref[...] = jnp.dot(emb, ar_ref[...], preferred_element_type=_f32)


def _emb_call(xp, wt, al16, ar16):
    bn = 512
    grid = (NP // bn,)
    return pl.pallas_call(
        _emb_body,
        grid=grid,
        in_specs=[
            pl.BlockSpec((bn, D), lambda i: (i, 0)),
            pl.BlockSpec((D, D), lambda i: (0, 0)),
            pl.BlockSpec((D, 16), lambda i: (0, 0)),
            pl.BlockSpec((D, 16), lambda i: (0, 0)),
        ],
        out_specs=[
            pl.BlockSpec((bn, DW), lambda i: (i, 0)),
            pl.BlockSpec((bn, 16), lambda i: (i, 0)),
        ],
        out_shape=[
            jax.ShapeDtypeStruct((NP, DW), _f32),
            jax.ShapeDtypeStruct((NP, 16), _f32),
        ],
    )(xp, wt, al16, ar16)


def _edge_body(src_hbm, dst_hbm, ext_hbm, r_hbm,
               outu_hbm,
               sd, rs, eb, acc_sh,
               isem, gsem, ssem, zsem):
    cid = lax.axis_index("c")
    sid = lax.axis_index("s")
    ebase = jnp.where(cid == 0, sid * EPT0, 16 * EPT0 + sid * EPT1)
    nck = jnp.where(cid == 0, EPT0 // CHUNK, EPT1 // CHUNK)

    def _zero_body(i, _):
        r = i // 9
        col = (i % 9) * 16
        eb[0, r, pl.ds(col, 16)] = jnp.zeros((16,), _f32)
        return 0
    lax.fori_loop(0, CHUNK * 9, _zero_body, 0)

    row0 = sid * RPT
    for j in range(RPT // CHUNK):
        pltpu.async_copy(eb.at[0], acc_sh.at[pl.ds(row0 + j * CHUNK, CHUNK)], zsem)
    for j in range(RPT // CHUNK):
        pltpu.make_async_copy(eb.at[0], acc_sh.at[pl.ds(row0 + j * CHUNK, CHUNK)], zsem).wait()
    plsc.subcore_barrier()

    def _fire_idx(ch, s):
        pltpu.async_copy(src_hbm.at[pl.ds(ebase + ch * CHUNK, CHUNK)],
                         sd.at[s, 0], isem.at[s])
        pltpu.async_copy(dst_hbm.at[pl.ds(ebase + ch * CHUNK, CHUNK)],
                         sd.at[s, 1], isem.at[s])

    def _wait_idx(ch, s):
        pltpu.make_async_copy(src_hbm.at[pl.ds(ebase + ch * CHUNK, CHUNK)],
                              sd.at[s, 0], isem.at[s]).wait()
        pltpu.make_async_copy(dst_hbm.at[pl.ds(ebase + ch * CHUNK, CHUNK)],
                              sd.at[s, 1], isem.at[s]).wait()

    def _fire(s, b):
        pltpu.async_copy(ext_hbm.at[sd.at[s, 0]], eb.at[b], gsem.at[b, 0])
        pltpu.async_copy(r_hbm.at[sd.at[s, 1]], rs.at[b], gsem.at[b, 1])

    def _wait(s, b):
        pltpu.make_async_copy(ext_hbm.at[sd.at[s, 0]], eb.at[b], gsem.at[b, 0]).wait()
        pltpu.make_async_copy(r_hbm.at[sd.at[s, 1]], rs.at[b], gsem.at[b, 1]).wait()

    def _compute(b):
        def _edge(e, _):
            x = eb[b, e, pl.ds(D, 16)] + rs[b, e]
            w = jnp.exp(jnp.maximum(x, 0.2 * x))
            eb[b, e, pl.ds(D, 16)] = w
            for h in range(H):
                sl = pl.ds(h * 16, 16)
                eb[b, e, sl] = eb[b, e, sl] * w[h]
            return 0
        lax.fori_loop(0, CHUNK, _edge, 0)

    def _scatter_fire(s, b):
        pltpu.async_copy(eb.at[b], acc_sh.at[sd.at[s, 1]], ssem.at[b], add=True)

    def _scatter_wait(s, b):
        pltpu.make_async_copy(eb.at[b], acc_sh.at[sd.at[s, 1]], ssem.at[b]).wait()

    for p in range(4):
        _fire_idx(p, p)
    _wait_idx(0, 0)
    _fire(0, 0)
    _wait_idx(1, 1)
    _fire(1, 1)

    def _group(g, _):
        ch0 = g * NSLOT
        for off in range(NSLOT):
            ch = ch0 + off
            s = off
            b = off % NBUF
            if off == 0:
                @pl.when(g > 0)
                def _():
                    _scatter_wait(NSLOT - 1, (NSLOT - 1) % NBUF)
            else:
                _scatter_wait(s - 1, (s - 1) % NBUF)

            @pl.when(ch + 4 < nck)
            def _():
                _fire_idx(ch + 4, (s + 4) % NSLOT)

            @pl.when(ch + 2 < nck)
            def _():
                _wait_idx(ch + 2, (s + 2) % NSLOT)
                _fire((s + 2) % NSLOT, (b + 2) % NBUF)
            _wait(s, b)
            _compute(b)
            _scatter_fire(s, b)
        return 0

    lax.fori_loop(0, nck // NSLOT, _group, 0)
    _scatter_wait(NSLOT - 1, (NSLOT - 1) % NBUF)
    plsc.subcore_barrier()

    pltpu.async_copy(acc_sh.at[pl.ds(row0, RPT)], outu_hbm.at[cid, pl.ds(row0, RPT)], zsem)
    pltpu.make_async_copy(acc_sh.at[pl.ds(row0, RPT)], outu_hbm.at[cid, pl.ds(row0, RPT)], zsem).wait()


def _edge_call(srcp, dstp, ext, r16):
    mesh = plsc.VectorSubcoreMesh(core_axis_name="c", subcore_axis_name="s")
    fn = pl.kernel(
        _edge_body,
        out_type=jax.ShapeDtypeStruct((2, NP, DW), _f32),
        mesh=mesh,
        scratch_types=(
            pltpu.VMEM((NSLOT, 2, CHUNK), jnp.int32),
            pltpu.VMEM((NBUF, CHUNK, 16), _f32),
            pltpu.VMEM((NBUF, CHUNK, DW), _f32),
            pltpu.VMEM_SHARED((NP, DW), _f32),
            pltpu.SemaphoreType.DMA((NSLOT,)),
            pltpu.SemaphoreType.DMA((NBUF, 2)),
            pltpu.SemaphoreType.DMA((NBUF,)),
            pltpu.SemaphoreType.DMA,
        ),
        compiler_params=pltpu.CompilerParams(use_tc_tiling_on_sc=False),
    )
    return fn(srcp, dstp, ext, r16)


def _fin_body(u0_ref, u1_ref, exp_ref, b_ref, o_ref):
    den = u0_ref[:, D:] + u1_ref[:, D:]
    dexp = jnp.dot(den, exp_ref[...], preferred_element_type=_f32)
    dsafe = jnp.where(dexp == 0.0, 1.0, dexp)
    o_ref[...] = (u0_ref[:, :D] + u1_ref[:, :D]) / dsafe + b_ref[...]


def _fin_call(u0, u1, expand, bias2d):
    bn = 512
    grid = (NP // bn,)
    return pl.pallas_call(
        _fin_body,
        grid=grid,
        in_specs=[
            pl.BlockSpec((bn, DW), lambda i: (i, 0)),
            pl.BlockSpec((bn, DW), lambda i: (i, 0)),
            pl.BlockSpec((16, D), lambda i: (0, 0)),
            pl.BlockSpec((1, D), lambda i: (0, 0)),
        ],
        out_specs=pl.BlockSpec((bn, D), lambda i: (i, 0)),
        out_shape=jax.ShapeDtypeStruct((NP, D), _f32),
    )(u0, u1, expand, bias2d)


def kernel(node_feats, edge_index, W, a_left, a_right, bias):
    xp = jnp.zeros((NP, D), _f32).at[:N].set(node_feats)
    src = edge_index[0].astype(jnp.int32)
    dst = edge_index[1].astype(jnp.int32)
    srcp = jnp.full((EPAD,), NP - 1, jnp.int32).at[:E].set(src)
    dstp = jnp.full((EPAD,), NP - 1, jnp.int32).at[:E].set(dst)

    rows = jnp.arange(D)[:, None] // C
    cols = jnp.arange(16)[None, :]
    al16 = jnp.where(cols == rows, a_left.T.reshape(D, 1), 0.0).astype(_f32)
    ar16 = jnp.where(cols == rows, a_right.T.reshape(D, 1), 0.0).astype(_f32)
    expand = (jnp.arange(16)[:, None] == (jnp.arange(D)[None, :] // C)).astype(_f32)
    bias2d = bias.reshape(1, D).astype(_f32)

    ext, r16 = _emb_call(xp, W.T.astype(_f32), al16, ar16)
    outu = _edge_call(srcp, dstp, ext, r16)
    res = _fin_call(outu[0], outu[1], expand, bias2d)
    return res[:N]


# speedup vs baseline: 1.4551x; 1.4551x over previous
import functools

import jax
import jax.numpy as jnp
from jax import lax
from jax.experimental import pallas as pl
from jax.experimental.pallas import tpu as pltpu
from jax.experimental.pallas import tpu_sc as plsc

N = 10000
E = 320000
D = 128
H = 8
C = 16

NP = 10240
CHUNK = 64
EPT0 = 14848
EPT1 = 5632
EPAD = 16 * (EPT0 + EPT1)
RPT = NP // 16

_f32 = jnp.float32


def _emb_body(x_ref, wt_ref, al_ref, ar_ref, emb_ref, l_ref, r_ref):
    emb = jnp.dot(x_ref[...], wt_ref[...], preferred_element_type=_f32)
    emb_ref[...] = emb
    l_ref[...] = jnp.dot(emb, al_ref[...], preferred_element_type=_f32)
    r_ref[...] = jnp.dot(emb, ar_ref[...], preferred_element_type=_f32)


def _emb_call(xp, wt, al16, ar16):
    bn = 512
    grid = (NP // bn,)
    return pl.pallas_call(
        _emb_body,
        grid=grid,
        in_specs=[
            pl.BlockSpec((bn, D), lambda i: (i, 0)),
            pl.BlockSpec((D, D), lambda i: (0, 0)),
            pl.BlockSpec((D, 16), lambda i: (0, 0)),
            pl.BlockSpec((D, 16), lambda i: (0, 0)),
        ],
        out_specs=[
            pl.BlockSpec((bn, D), lambda i: (i, 0)),
            pl.BlockSpec((bn, 16), lambda i: (i, 0)),
            pl.BlockSpec((bn, 16), lambda i: (i, 0)),
        ],
        out_shape=[
            jax.ShapeDtypeStruct((NP, D), _f32),
            jax.ShapeDtypeStruct((NP, 16), _f32),
            jax.ShapeDtypeStruct((NP, 16), _f32),
        ],
    )(xp, wt, al16, ar16)


def _edge_body(src_hbm, dst_hbm, emb_hbm, l_hbm, r_hbm,
               outu_hbm, den_hbm,
               sd, ls, rs, wb, eb, acc_sh, den_sh,
               isem, gsem, ssem):
    cid = lax.axis_index("c")
    sid = lax.axis_index("s")
    ebase = jnp.where(cid == 0, sid * EPT0, 16 * EPT0 + sid * EPT1)
    nck = jnp.where(cid == 0, EPT0 // CHUNK, EPT1 // CHUNK)

    def _zero_body(i, _):
        r = i // 8
        col = (i % 8) * 16
        eb[0, r, pl.ds(col, 16)] = jnp.zeros((16,), _f32)
        return 0
    lax.fori_loop(0, CHUNK * 8, _zero_body, 0)

    def _zero16_body(i, _):
        wb[0, i] = jnp.zeros((16,), _f32)
        return 0
    lax.fori_loop(0, CHUNK, _zero16_body, 0)

    row0 = sid * RPT
    for j in range(RPT // CHUNK):
        pltpu.async_copy(eb.at[0], acc_sh.at[pl.ds(row0 + j * CHUNK, CHUNK)], ssem.at[0, 1])
        pltpu.async_copy(wb.at[0], den_sh.at[pl.ds(row0 + j * CHUNK, CHUNK)], ssem.at[0, 0])
    for j in range(RPT // CHUNK):
        pltpu.make_async_copy(eb.at[0], acc_sh.at[pl.ds(row0 + j * CHUNK, CHUNK)], ssem.at[0, 1]).wait()
        pltpu.make_async_copy(wb.at[0], den_sh.at[pl.ds(row0 + j * CHUNK, CHUNK)], ssem.at[0, 0]).wait()
    plsc.subcore_barrier()

    def _fire_idx(ch, s):
        pltpu.async_copy(src_hbm.at[pl.ds(ebase + ch * CHUNK, CHUNK)],
                         sd.at[s, 0], isem.at[s])
        pltpu.async_copy(dst_hbm.at[pl.ds(ebase + ch * CHUNK, CHUNK)],
                         sd.at[s, 1], isem.at[s])

    def _wait_idx(ch, s):
        pltpu.make_async_copy(src_hbm.at[pl.ds(ebase + ch * CHUNK, CHUNK)],
                              sd.at[s, 0], isem.at[s]).wait()
        pltpu.make_async_copy(dst_hbm.at[pl.ds(ebase + ch * CHUNK, CHUNK)],
                              sd.at[s, 1], isem.at[s]).wait()

    def _fire(s, b):
        pltpu.async_copy(l_hbm.at[sd.at[s, 0]], ls.at[b], gsem.at[b, 0])
        pltpu.async_copy(r_hbm.at[sd.at[s, 1]], rs.at[b], gsem.at[b, 1])
        pltpu.async_copy(emb_hbm.at[sd.at[s, 0]], eb.at[b], gsem.at[b, 2])

    def _wait(s, b):
        pltpu.make_async_copy(l_hbm.at[sd.at[s, 0]], ls.at[b], gsem.at[b, 0]).wait()
        pltpu.make_async_copy(r_hbm.at[sd.at[s, 1]], rs.at[b], gsem.at[b, 1]).wait()
        pltpu.make_async_copy(emb_hbm.at[sd.at[s, 0]], eb.at[b], gsem.at[b, 2]).wait()

    def _compute(b):
        def _edge(e, _):
            x = ls[b, e] + rs[b, e]
            w = jnp.exp(jnp.maximum(x, 0.2 * x))
            wb[b, e] = w
            for h in range(H):
                sl = pl.ds(h * 16, 16)
                eb[b, e, sl] = eb[b, e, sl] * w[h]
            return 0
        lax.fori_loop(0, CHUNK, _edge, 0)

    def _scatter_fire(s, b):
        pltpu.async_copy(wb.at[b], den_sh.at[sd.at[s, 1]], ssem.at[b, 0], add=True)
        pltpu.async_copy(eb.at[b], acc_sh.at[sd.at[s, 1]], ssem.at[b, 1], add=True)

    def _scatter_wait(s, b):
        pltpu.make_async_copy(wb.at[b], den_sh.at[sd.at[s, 1]], ssem.at[b, 0]).wait()
        pltpu.make_async_copy(eb.at[b], acc_sh.at[sd.at[s, 1]], ssem.at[b, 1]).wait()

    _fire_idx(0, 0)
    _fire_idx(1, 1)
    _wait_idx(0, 0)
    _fire(0, 0)

    def _quad(q, _):
        ch0 = q * 4
        for off in range(4):
            ch = ch0 + off
            s = off
            b = off % 2
            if off == 0:
                @pl.when(q > 0)
                def _():
                    _scatter_wait(3, 1)
            else:
                _scatter_wait(s - 1, 1 - b)
            _wait(s, b)
            if off < 2:
                @pl.when(ch + 2 < nck)
                def _():
                    _fire_idx(ch + 2, s + 2)
                _wait_idx(ch + 1, s + 1)
                _fire(s + 1, 1 - b)
            else:
                @pl.when(ch + 2 < nck)
                def _():
                    _fire_idx(ch + 2, s - 2)

                @pl.when(ch + 1 < nck)
                def _():
                    _wait_idx(ch + 1, (s + 1) % 4)
                    _fire((s + 1) % 4, 1 - b)
            _compute(b)
            _scatter_fire(s, b)
        return 0

    lax.fori_loop(0, nck // 4, _quad, 0)
    _scatter_wait(3, 1)
    plsc.subcore_barrier()

    pltpu.async_copy(acc_sh.at[pl.ds(row0, RPT)], outu_hbm.at[cid, pl.ds(row0, RPT)], ssem.at[0, 1])
    pltpu.async_copy(den_sh.at[pl.ds(row0, RPT)], den_hbm.at[cid, pl.ds(row0, RPT)], ssem.at[0, 0])
    pltpu.make_async_copy(acc_sh.at[pl.ds(row0, RPT)], outu_hbm.at[cid, pl.ds(row0, RPT)], ssem.at[0, 1]).wait()
    pltpu.make_async_copy(den_sh.at[pl.ds(row0, RPT)], den_hbm.at[cid, pl.ds(row0, RPT)], ssem.at[0, 0]).wait()


def _edge_call(srcp, dstp, emb, l16, r16):
    mesh = plsc.VectorSubcoreMesh(core_axis_name="c", subcore_axis_name="s")
    fn = pl.kernel(
        _edge_body,
        out_type=(
            jax.ShapeDtypeStruct((2, NP, D), _f32),
            jax.ShapeDtypeStruct((2, NP, 16), _f32),
        ),
        mesh=mesh,
        scratch_types=(
            pltpu.VMEM((4, 2, CHUNK), jnp.int32),
            pltpu.VMEM((2, CHUNK, 16), _f32),
            pltpu.VMEM((2, CHUNK, 16), _f32),
            pltpu.VMEM((2, CHUNK, 16), _f32),
            pltpu.VMEM((2, CHUNK, D), _f32),
            pltpu.VMEM_SHARED((NP, D), _f32),
            pltpu.VMEM_SHARED((NP, 16), _f32),
            pltpu.SemaphoreType.DMA((4,)),
            pltpu.SemaphoreType.DMA((2, 3)),
            pltpu.SemaphoreType.DMA((2, 2)),
        ),
        compiler_params=pltpu.CompilerParams(use_tc_tiling_on_sc=False),
    )
    return fn(srcp, dstp, emb, l16, r16)


def _fin_body(u0_ref, u1_ref, d0_ref, d1_ref, exp_ref, b_ref, o_ref):
    den = d0_ref[...] + d1_ref[...]
    dexp = jnp.dot(den, exp_ref[...], preferred_element_type=_f32)
    dsafe = jnp.where(dexp == 0.0, 1.0, dexp)
    o_ref[...] = (u0_ref[...] + u1_ref[...]) / dsafe + b_ref[...]


def _fin_call(u0, u1, d0, d1, expand, bias2d):
    bn = 512
    grid = (NP // bn,)
    return pl.pallas_call(
        _fin_body,
        grid=grid,
        in_specs=[
            pl.BlockSpec((bn, D), lambda i: (i, 0)),
            pl.BlockSpec((bn, D), lambda i: (i, 0)),
            pl.BlockSpec((bn, 16), lambda i: (i, 0)),
            pl.BlockSpec((bn, 16), lambda i: (i, 0)),
            pl.BlockSpec((16, D), lambda i: (0, 0)),
            pl.BlockSpec((1, D), lambda i: (0, 0)),
        ],
        out_specs=pl.BlockSpec((bn, D), lambda i: (i, 0)),
        out_shape=jax.ShapeDtypeStruct((NP, D), _f32),
    )(u0, u1, d0, d1, expand, bias2d)


def kernel(node_feats, edge_index, W, a_left, a_right, bias):
    xp = jnp.zeros((NP, D), _f32).at[:N].set(node_feats)
    src = edge_index[0].astype(jnp.int32)
    dst = edge_index[1].astype(jnp.int32)
    srcp = jnp.full((EPAD,), NP - 1, jnp.int32).at[:E].set(src)
    dstp = jnp.full((EPAD,), NP - 1, jnp.int32).at[:E].set(dst)

    rows = jnp.arange(D)[:, None] // C
    cols = jnp.arange(16)[None, :]
    al16 = jnp.where(cols == rows, a_left.T.reshape(D, 1), 0.0).astype(_f32)
    ar16 = jnp.where(cols == rows, a_right.T.reshape(D, 1), 0.0).astype(_f32)
    expand = (jnp.arange(16)[:, None] == (jnp.arange(D)[None, :] // C)).astype(_f32)
    bias2d = bias.reshape(1, D).astype(_f32)

    emb, l16, r16 = _emb_call(xp, W.T.astype(_f32), al16, ar16)
    outu, den = _edge_call(srcp, dstp, emb, l16, r16)
    res = _fin_call(outu[0], outu[1], den[0], den[1], expand, bias2d)
    return res[:N]
